# baseline (speedup 1.0000x reference)
import jax
import jax.numpy as jnp
from jax import lax
from jax.experimental import pallas as pl
from jax.experimental.pallas import tpu as pltpu
from jax.experimental.pallas import tpu_sc as plsc

HIDDEN = 1024
N = 4 * 4096
NC, NS, L = 2, 16, 16
NW = NC * NS
RPW = N // NW
C = 8
NCHUNK = RPW // C
D = 4


def _emb_body(w_hbm, p_hbm, wi_hbm, pi_hbm, o_hbm,
              widx, pidx,
              wb0, wb1, wb2, wb3, pb0, pb1, pb2, pb3,
              ob0, ob1, ob2, ob3,
              sw0, sw1, sw2, sw3, sp0, sp1, sp2, sp3,
              ss0, ss1, ss2, ss3):
    wbuf = (wb0, wb1, wb2, wb3)
    pbuf = (pb0, pb1, pb2, pb3)
    obuf = (ob0, ob1, ob2, ob3)
    sem_w = (sw0, sw1, sw2, sw3)
    sem_p = (sp0, sp1, sp2, sp3)
    sem_s = (ss0, ss1, ss2, ss3)

    wid = lax.axis_index("s") * NC + lax.axis_index("c")
    base = wid * RPW
    pltpu.sync_copy(wi_hbm.at[pl.ds(base, RPW)], widx)
    pltpu.sync_copy(pi_hbm.at[pl.ds(base, RPW)], pidx)

    def gather_copies(g, b):
        cw = pltpu.make_async_copy(
            w_hbm.at[widx.at[pl.ds(g * C, C)]], wbuf[b], sem_w[b])
        cp = pltpu.make_async_copy(
            p_hbm.at[pidx.at[pl.ds(g * C, C)]], pbuf[b], sem_p[b])
        return cw, cp

    def store_copy(g, b):
        return pltpu.make_async_copy(
            obuf[b], o_hbm.at[pl.ds(base + g * C, C)], sem_s[b])

    for b in range(D):
        cw, cp = gather_copies(b, b)
        cw.start()
        cp.start()

    @pl.loop(0, NCHUNK, step=D)
    def _quad(g):
        for b in range(D):
            gg = g + b
            cw, cp = gather_copies(gg, b)
            cw.wait()
            cp.wait()

            @pl.when(gg >= D)
            def _():
                store_copy(gg - D, b).wait()

            @pl.loop(0, C)
            def _row(r):
                for u in range(HIDDEN // L):
                    s = pl.ds(u * L, L)
                    obuf[b][r, s] = wbuf[b][r, s] + pbuf[b][r, s]

            store_copy(gg, b).start()

            @pl.when(gg + D < NCHUNK)
            def _():
                nw, np_ = gather_copies(gg + D, b)
                nw.start()
                np_.start()

    for b in range(D):
        store_copy(NCHUNK - D + b, b).wait()


def kernel(input_ids, position_ids, word_embeddings, position_embeddings):
    wids = input_ids.reshape(-1).astype(jnp.int32)
    pids = position_ids.reshape(-1).astype(jnp.int32)
    mesh = plsc.VectorSubcoreMesh(core_axis_name="c", subcore_axis_name="s")
    k = pl.kernel(
        _emb_body,
        out_type=jax.ShapeDtypeStruct((N, HIDDEN), jnp.float32),
        mesh=mesh,
        scratch_types=(
            [pltpu.VMEM((RPW,), jnp.int32)] * 2
            + [pltpu.VMEM((C, HIDDEN), jnp.float32)] * (3 * D)
            + [pltpu.SemaphoreType.DMA] * (3 * D)
        ),
    )
    out = k(word_embeddings, position_embeddings, wids, pids)
    return out.reshape(input_ids.shape + (HIDDEN,))
